# Initial kernel scaffold; baseline (speedup 1.0000x reference)
import jax
import jax.numpy as jnp
from jax.experimental import pallas as pl


def kernel(x, x_mask, W):
    raise NotImplementedError("write your pallas kernel here")


# speedup vs baseline: 1.0617x; 1.0617x over previous
import jax
import jax.numpy as jnp
from jax.experimental import pallas as pl
from jax.experimental.pallas import tpu as pltpu


_TOP_K = 8
_BLOCK_T = 512


def _router_block(x_ref, m_ref, w_ref, logits_ref, probs_ref, wts_ref, idx_ref):
    x = x_ref[...]
    w = w_ref[...]
    m = m_ref[...]
    bt = x.shape[0]
    e = w.shape[0]

    raw = jax.lax.dot_general(
        x, w, (((1,), (1,)), ((), ())), preferred_element_type=jnp.float32
    )
    logits = raw * (m * m)
    logits_ref[...] = logits

    mx = jnp.max(logits, axis=-1, keepdims=True)
    ex = jnp.exp(logits - mx)
    sm = ex / jnp.sum(ex, axis=-1, keepdims=True)
    probs_ref[...] = sm * m

    iota = jax.lax.broadcasted_iota(jnp.int32, (bt, e), 1)
    cur = sm
    vals = []
    idxs = []
    for _ in range(_TOP_K):
        v = jnp.max(cur, axis=-1, keepdims=True)
        cand = jnp.where(cur >= v, iota, e)
        ix = jnp.min(cand, axis=-1, keepdims=True)
        vals.append(v)
        idxs.append(ix)
        cur = jnp.where(iota == ix, -jnp.inf, cur)
    wv = jnp.concatenate(vals, axis=-1)
    iv = jnp.concatenate(idxs, axis=-1)

    s = jnp.sum(wv, axis=-1, keepdims=True)
    wv = wv / jnp.where(s > 0, s, jnp.ones_like(s))
    wts_ref[...] = wv * m
    idx_ref[...] = jnp.where(m != 0.0, iv, -1)


def kernel(x, x_mask, W):
    b, t, c = x.shape
    e = W.shape[0]
    n = b * t
    x2 = x.reshape(n, c)
    m2 = x_mask.reshape(n, 1)

    grid = (n // _BLOCK_T,)
    logits, probs, wts, idx = pl.pallas_call(
        _router_block,
        grid=grid,
        in_specs=[
            pl.BlockSpec((_BLOCK_T, c), lambda i: (i, 0)),
            pl.BlockSpec((_BLOCK_T, 1), lambda i: (i, 0)),
            pl.BlockSpec((e, c), lambda i: (0, 0)),
        ],
        out_specs=[
            pl.BlockSpec((_BLOCK_T, e), lambda i: (i, 0)),
            pl.BlockSpec((_BLOCK_T, e), lambda i: (i, 0)),
            pl.BlockSpec((_BLOCK_T, _TOP_K), lambda i: (i, 0)),
            pl.BlockSpec((_BLOCK_T, _TOP_K), lambda i: (i, 0)),
        ],
        out_shape=[
            jax.ShapeDtypeStruct((n, e), jnp.float32),
            jax.ShapeDtypeStruct((n, e), jnp.float32),
            jax.ShapeDtypeStruct((n, _TOP_K), jnp.float32),
            jax.ShapeDtypeStruct((n, _TOP_K), jnp.int32),
        ],
        compiler_params=pltpu.CompilerParams(
            dimension_semantics=("arbitrary",),
        ),
    )(x2, m2, W)

    return (
        wts.reshape(b, t, _TOP_K),
        idx.reshape(b, t, _TOP_K),
        logits.reshape(b, t, e),
        probs.reshape(b, t, e),
    )


# speedup vs baseline: 1.5964x; 1.5036x over previous
import jax
import jax.numpy as jnp
from jax.experimental import pallas as pl
from jax.experimental.pallas import tpu as pltpu


_TOP_K = 8
_BLOCK_T = 512


def _router_block(x_ref, m_ref, w_ref, logits_ref, probs_ref, wts_ref, idx_ref):
    x = x_ref[...]
    w = w_ref[...]
    m = m_ref[...]
    e = w.shape[0]
    bt = x.shape[0]

    raw = jax.lax.dot_general(
        w, x, (((1,), (1,)), ((), ())), preferred_element_type=jnp.float32
    )
    logits = raw * (m * m)
    logits_ref[...] = logits

    mx = jnp.max(logits, axis=0, keepdims=True)
    ex = jnp.exp(logits - mx)
    sm = ex / jnp.sum(ex, axis=0, keepdims=True)
    probs_ref[...] = sm * m

    iota = jax.lax.broadcasted_iota(jnp.int32, (e, bt), 0)
    key = ((sm.view(jnp.int32) & jnp.int32(~63)) | (jnp.int32(e - 1) - iota))
    vals = []
    idxs = []
    for _ in range(_TOP_K):
        c = jnp.max(key, axis=0, keepdims=True)
        vals.append((c & jnp.int32(~63)).view(jnp.float32))
        idxs.append(jnp.int32(e - 1) - (c & jnp.int32(63)))
        key = jnp.where(key == c, jnp.int32(-1), key)
    wv = jnp.concatenate(vals, axis=0)
    iv = jnp.concatenate(idxs, axis=0)

    s = jnp.sum(wv, axis=0, keepdims=True)
    wv = wv / jnp.where(s > 0, s, jnp.ones_like(s))
    wts_ref[...] = wv * m
    idx_ref[...] = jnp.where(m != 0.0, iv, -1)


def kernel(x, x_mask, W):
    b, t, c = x.shape
    e = W.shape[0]
    n = b * t
    x2 = x.reshape(n, c)
    m2 = x_mask.reshape(1, n)

    grid = (n // _BLOCK_T,)
    logits_t, probs_t, wts_t, idx_t = pl.pallas_call(
        _router_block,
        grid=grid,
        in_specs=[
            pl.BlockSpec((_BLOCK_T, c), lambda i: (i, 0)),
            pl.BlockSpec((1, _BLOCK_T), lambda i: (0, i)),
            pl.BlockSpec((e, c), lambda i: (0, 0)),
        ],
        out_specs=[
            pl.BlockSpec((e, _BLOCK_T), lambda i: (0, i)),
            pl.BlockSpec((e, _BLOCK_T), lambda i: (0, i)),
            pl.BlockSpec((_TOP_K, _BLOCK_T), lambda i: (0, i)),
            pl.BlockSpec((_TOP_K, _BLOCK_T), lambda i: (0, i)),
        ],
        out_shape=[
            jax.ShapeDtypeStruct((e, n), jnp.float32),
            jax.ShapeDtypeStruct((e, n), jnp.float32),
            jax.ShapeDtypeStruct((_TOP_K, n), jnp.float32),
            jax.ShapeDtypeStruct((_TOP_K, n), jnp.int32),
        ],
        compiler_params=pltpu.CompilerParams(
            dimension_semantics=("arbitrary",),
        ),
    )(x2, m2, W)

    return (
        wts_t.T.reshape(b, t, _TOP_K),
        idx_t.T.reshape(b, t, _TOP_K),
        logits_t.T.reshape(b, t, e),
        probs_t.T.reshape(b, t, e),
    )
